# baseline (speedup 1.0000x reference)
import jax
import jax.numpy as jnp
from jax import lax
from jax.experimental import pallas as pl
from jax.experimental.pallas import tpu as pltpu
from jax.experimental.pallas import tpu_sc as plsc

N_NODES = 100000
NUM_TYPES = 128
LANES = 16
CHUNK = 400
NCHUNKS = N_NODES // CHUNK
FLAT = CHUNK * NUM_TYPES
GROUPS = CHUNK // LANES

try:
    _info = plsc.get_sparse_core_info()
    _NC = _info.num_cores
    _NW = _NC * _info.num_subcores
except Exception:
    _NC = 2
    _NW = 32
_BASE_STEPS = NCHUNKS // _NW
_EXTRA = NCHUNKS - _BASE_STEPS * _NW

_mesh = plsc.VectorSubcoreMesh(core_axis_name="c", subcore_axis_name="s")


_MAX_STEPS = _BASE_STEPS + 1


def _scratch_types():
    return [
        pltpu.VMEM((CHUNK, NUM_TYPES), jnp.float32),
        pltpu.VMEM((CHUNK, NUM_TYPES), jnp.float32),
        pltpu.VMEM((_MAX_STEPS * CHUNK,), jnp.int32),
        pltpu.SemaphoreType.DMA,
        pltpu.SemaphoreType.DMA,
        pltpu.SemaphoreType.DMA,
    ]


def _onehot_body(atoms_hbm, out_hbm, buf0, buf1, idxall, sem0, sem1, sem_i):
    wid = lax.axis_index("s") * _NC + lax.axis_index("c")
    lane = lax.iota(jnp.int32, LANES)
    ones = jnp.full((LANES,), 1.0, jnp.float32)
    zeros = jnp.zeros((LANES,), jnp.float32)

    bufs = (buf0, buf1)
    sems = (sem0, sem1)

    def scatter(buf, step, val):
        def _s(g, carry):
            iv = idxall[pl.ds(step * CHUNK + g * LANES, LANES)]
            rows = lane + g * LANES
            plsc.store_scatter(buf, [rows, iv], val)
            return carry

        lax.fori_loop(0, GROUPS, _s, 0, unroll=5)

    def emit(nsteps):
        idx_dmas = []
        for i in range(nsteps):
            c = wid + i * _NW
            idx_dmas.append(
                pltpu.async_copy(
                    atoms_hbm.at[pl.ds(c * CHUNK, CHUNK)],
                    idxall.at[pl.ds(i * CHUNK, CHUNK)],
                    sem_i,
                )
            )

        def _zero(r, carry):
            for j in range(NUM_TYPES // LANES):
                buf0[r, pl.ds(j * LANES, LANES)] = zeros
                buf1[r, pl.ds(j * LANES, LANES)] = zeros
            return carry

        lax.fori_loop(0, CHUNK, _zero, 0, unroll=2)
        for h in idx_dmas:
            h.wait()

        pending = [None, None]
        for i in range(nsteps):
            b = i % 2
            c = wid + i * _NW
            if pending[b] is not None:
                pending[b].wait()
                scatter(bufs[b], i - 2, zeros)
            scatter(bufs[b], i, ones)
            pending[b] = pltpu.async_copy(
                bufs[b], out_hbm.at[pl.ds(c * CHUNK, CHUNK)], sems[b]
            )
        for b in range(2):
            if pending[b] is not None:
                pending[b].wait()

    @pl.when(wid < _EXTRA)
    def _():
        emit(_BASE_STEPS + 1)

    @pl.when(wid >= _EXTRA)
    def _():
        emit(_BASE_STEPS)


_onehot = pl.kernel(
    _onehot_body,
    mesh=_mesh,
    compiler_params=pltpu.CompilerParams(needs_layout_passes=False),
    out_type=jax.ShapeDtypeStruct((N_NODES, NUM_TYPES), jnp.float32),
    scratch_types=_scratch_types(),
)


def kernel(atom_types):
    return _onehot(atom_types.astype(jnp.int32))


# speedup vs baseline: 1.0448x; 1.0448x over previous
import jax
import jax.numpy as jnp
from jax import lax
from jax.experimental import pallas as pl
from jax.experimental.pallas import tpu as pltpu
from jax.experimental.pallas import tpu_sc as plsc

N_NODES = 100000
NUM_TYPES = 128
LANES = 16
CHUNK = 400
NCHUNKS = N_NODES // CHUNK
FLAT = CHUNK * NUM_TYPES
GROUPS = CHUNK // LANES

try:
    _info = plsc.get_sparse_core_info()
    _NC = _info.num_cores
    _NW = _NC * _info.num_subcores
except Exception:
    _NC = 2
    _NW = 32
_BASE_STEPS = NCHUNKS // _NW
_EXTRA = NCHUNKS - _BASE_STEPS * _NW

_mesh = plsc.VectorSubcoreMesh(core_axis_name="c", subcore_axis_name="s")


_MAX_STEPS = _BASE_STEPS + 1


def _scratch_types():
    return [
        pltpu.VMEM((CHUNK, NUM_TYPES), jnp.float32),
        pltpu.VMEM((CHUNK, NUM_TYPES), jnp.float32),
        pltpu.VMEM((_MAX_STEPS * CHUNK,), jnp.int32),
        pltpu.SemaphoreType.DMA,
        pltpu.SemaphoreType.DMA,
        pltpu.SemaphoreType.DMA,
    ]


def _onehot_body(atoms_hbm, out_hbm, buf0, buf1, idxall, sem0, sem1, sem_i):
    wid = lax.axis_index("s") * _NC + lax.axis_index("c")
    lane = lax.iota(jnp.int32, LANES)
    ones = jnp.full((LANES,), 1.0, jnp.float32)
    zeros = jnp.zeros((LANES,), jnp.float32)

    bufs = (buf0, buf1)
    sems = (sem0, sem1)

    def scatter(buf, step, val):
        def _s(g, carry):
            iv = idxall[pl.ds(step * CHUNK + g * LANES, LANES)]
            rows = lane + g * LANES
            plsc.store_scatter(buf, [rows, iv], val)
            return carry

        lax.fori_loop(0, GROUPS, _s, 0, unroll=5)

    def zero(buf):
        def _zero(r, carry):
            for j in range(NUM_TYPES // LANES):
                buf[r, pl.ds(j * LANES, LANES)] = zeros
            return carry

        lax.fori_loop(0, CHUNK, _zero, 0, unroll=2)

    def emit(nsteps):
        idx_dmas = []
        for i in range(nsteps):
            c = wid + i * _NW
            idx_dmas.append(
                pltpu.async_copy(
                    atoms_hbm.at[pl.ds(c * CHUNK, CHUNK)],
                    idxall.at[pl.ds(i * CHUNK, CHUNK)],
                    sem_i,
                )
            )

        pending = [None, None]

        def fill(i):
            b = i % 2
            c = wid + i * _NW
            idx_dmas[i].wait()
            if pending[b] is not None:
                pending[b].wait()
                scatter(bufs[b], i - 2, zeros)
            scatter(bufs[b], i, ones)
            pending[b] = pltpu.async_copy(
                bufs[b], out_hbm.at[pl.ds(c * CHUNK, CHUNK)], sems[b]
            )

        zero(buf0)
        fill(0)
        zero(buf1)
        for i in range(1, nsteps):
            fill(i)
        for b in range(2):
            if pending[b] is not None:
                pending[b].wait()

    @pl.when(wid < _EXTRA)
    def _():
        emit(_BASE_STEPS + 1)

    @pl.when(wid >= _EXTRA)
    def _():
        emit(_BASE_STEPS)


_onehot = pl.kernel(
    _onehot_body,
    mesh=_mesh,
    compiler_params=pltpu.CompilerParams(needs_layout_passes=False),
    out_type=jax.ShapeDtypeStruct((N_NODES, NUM_TYPES), jnp.float32),
    scratch_types=_scratch_types(),
)


def kernel(atom_types):
    return _onehot(atom_types.astype(jnp.int32))
